# Initial kernel scaffold; baseline (speedup 1.0000x reference)
import jax
import jax.numpy as jnp
from jax.experimental import pallas as pl


def kernel(x, params, edge_index, batch):
    raise NotImplementedError("write your pallas kernel here")


# speedup vs baseline: 3.0845x; 3.0845x over previous
import functools

import jax
import jax.numpy as jnp
from jax import lax
from jax.experimental import pallas as pl
from jax.experimental.pallas import tpu as pltpu
from jax.experimental.pallas import tpu_sc as plsc

_N, _D, _FD, _T, _G = 10000, 128, 32, 16, 32
_E = 320000
_NC, _NS = 2, 16
_NW = _NC * _NS
_K = 128
_C = 80
_EPAD = _NW * _C * _K
_NP = 10112
_RPT = _NP // _NS


@functools.cache
def _make_seg_sum():
    mesh = plsc.VectorSubcoreMesh(core_axis_name="c", subcore_axis_name="s",
                                  num_cores=_NC, num_subcores=_NS)

    @functools.partial(
        pl.kernel,
        mesh=mesh,
        out_type=jax.ShapeDtypeStruct((_NC, _NP, _D), jnp.float32),
        scratch_types=[
            pltpu.VMEM((_C, _K), jnp.int32),
            pltpu.VMEM((_C, _K), jnp.int32),
            pltpu.VMEM((_K, _D), jnp.float32),
            pltpu.VMEM_SHARED((_NP, _D), jnp.float32),
            pltpu.SemaphoreType.DMA,
        ],
    )
    def seg_sum(x_hbm, src_hbm, dst_hbm, zero_hbm, out_hbm,
                src_v, dst_v, rows_v, acc, sem):
        c = lax.axis_index("c")
        s = lax.axis_index("s")
        wid = s * _NC + c
        r0 = s * _RPT
        pltpu.sync_copy(zero_hbm.at[pl.ds(r0, _RPT)], acc.at[pl.ds(r0, _RPT)])
        pltpu.sync_copy(src_hbm.at[wid], src_v)
        pltpu.sync_copy(dst_hbm.at[wid], dst_v)
        plsc.subcore_barrier()

        def body(j, carry):
            pltpu.async_copy(x_hbm.at[src_v.at[j]], rows_v, sem).wait()
            pltpu.sync_copy(rows_v, acc.at[dst_v.at[j]], add=True)
            return carry

        lax.fori_loop(0, _C, body, 0)
        plsc.subcore_barrier()
        pltpu.sync_copy(acc.at[pl.ds(r0, _RPT)],
                        out_hbm.at[c].at[pl.ds(r0, _RPT)])

    return seg_sum


def _seg_sum(x_pad, srcp, dstp, zero_np):
    return _make_seg_sum()(x_pad, srcp, dstp, zero_np)


def _elu(v):
    return jnp.where(v > 0, v, jnp.exp(jnp.minimum(v, 0.0)) - 1.0)


def _bn(y, g, be):
    m = jnp.mean(y, axis=0, keepdims=True)
    v = jnp.mean((y - m) ** 2, axis=0, keepdims=True)
    return (y - m) / jnp.sqrt(v + 1e-5) * g + be


def _lin_block(h, W1, b1, g1, be1, W2, b2, g2, be2):
    a = jnp.dot(h, W1, preferred_element_type=jnp.float32) + b1
    e = _elu(_bn(a, g1, be1))
    cc = jnp.dot(e, W2, preferred_element_type=jnp.float32) + b2
    return _bn(cc, g2, be2)


def _dense1_body(x_ref, agg_ref, Wg, bg, gg, beg,
                 W1, b1, g1, be1, W2, b2, g2, be2,
                 h1_ref, z0_ref):
    x = x_ref[...]
    h2 = x + agg_ref[0, :_N, :] + agg_ref[1, :_N, :]
    y = jnp.dot(h2, Wg[...], preferred_element_type=jnp.float32) + bg[...]
    h1_ref[:_N, :] = _elu(_bn(y, gg[...], beg[...]))
    h1_ref[_N:, :] = jnp.zeros((_NP - _N, _D), jnp.float32)
    z0_ref[...] = _lin_block(x, W1[...], b1[...], g1[...], be1[...],
                             W2[...], b2[...], g2[...], be2[...])


_dense1 = pl.pallas_call(
    _dense1_body,
    out_shape=[
        jax.ShapeDtypeStruct((_NP, _D), jnp.float32),
        jax.ShapeDtypeStruct((_N, _T), jnp.float32),
    ],
)


def _dense2_body(h1p_ref, agg_ref, z0_ref, batch_ref,
                 Wg, bg, gg, beg,
                 aW1, ab1, ag1, abe1, aW2, ab2, ag2, abe2,
                 bW1, bb1, bg1, bbe1, bW2, bb2, bg2, bbe2,
                 out_ref, Z_ref, h_ref):
    h1 = h1p_ref[:_N, :]
    hpre = h1 + agg_ref[0, :_N, :] + agg_ref[1, :_N, :]
    y = jnp.dot(hpre, Wg[...], preferred_element_type=jnp.float32) + bg[...]
    hh = _elu(_bn(y, gg[...], beg[...]))
    h_ref[...] = hh
    z1 = _lin_block(h1, aW1[...], ab1[...], ag1[...], abe1[...],
                    aW2[...], ab2[...], ag2[...], abe2[...])
    z2 = _lin_block(hh, bW1[...], bb1[...], bg1[...], bbe1[...],
                    bW2[...], bb2[...], bg2[...], bbe2[...])
    z0 = z0_ref[...]
    Z_ref[...] = z0 + z1 + z2
    b = batch_ref[...]
    ninf = jnp.float32(-jnp.inf)
    zc = jnp.concatenate([z0, z1, z2], axis=1)

    def pool_body(g, carry):
        m = b == g
        p = jnp.max(jnp.where(m, zc, ninf), axis=0, keepdims=True)
        out_ref[pl.ds(g, 1), :] = p[:, 0:_T] + p[:, _T:2 * _T] + p[:, 2 * _T:]
        return carry

    lax.fori_loop(0, _G, pool_body, 0)


_dense2 = pl.pallas_call(
    _dense2_body,
    out_shape=[
        jax.ShapeDtypeStruct((_G, _T), jnp.float32),
        jax.ShapeDtypeStruct((_N, _T), jnp.float32),
        jax.ShapeDtypeStruct((_N, _D), jnp.float32),
    ],
)


def _flat_lin(p):
    return (p["W1"], p["b1"].reshape(1, -1), p["g1"].reshape(1, -1),
            p["be1"].reshape(1, -1), p["W2"], p["b2"].reshape(1, -1),
            p["g2"].reshape(1, -1), p["be2"].reshape(1, -1))


def _flat_gin(p):
    return (p["W"], p["b"].reshape(1, -1), p["g"].reshape(1, -1),
            p["be"].reshape(1, -1))


def kernel(x, params, edge_index, batch):
    pad = _EPAD - _E
    srcp = jnp.concatenate(
        [edge_index[0], jnp.full((pad,), _N, jnp.int32)]).reshape(_NW, _C, _K)
    dstp = jnp.concatenate(
        [edge_index[1], jnp.full((pad,), _N, jnp.int32)]).reshape(_NW, _C, _K)
    zero_np = jnp.zeros((_NP, _D), jnp.float32)
    x_pad = jnp.concatenate(
        [x, jnp.zeros((_NP - _N, _D), jnp.float32)], axis=0)

    agg1 = _seg_sum(x_pad, srcp, dstp, zero_np)
    h1p, z0 = _dense1(x, agg1, *_flat_gin(params["gin1"]),
                      *_flat_lin(params["lin0"]))
    agg2 = _seg_sum(h1p, srcp, dstp, zero_np)
    out, Z, h = _dense2(h1p, agg2, z0, batch.reshape(_N, 1),
                        *_flat_gin(params["gin2"]),
                        *_flat_lin(params["lin1"]),
                        *_flat_lin(params["lin2"]))
    return out, Z, h


# speedup vs baseline: 3.5603x; 1.1542x over previous
import functools

import jax
import jax.numpy as jnp
from jax import lax
from jax.experimental import pallas as pl
from jax.experimental.pallas import tpu as pltpu
from jax.experimental.pallas import tpu_sc as plsc

_N, _D, _FD, _T, _G = 10000, 128, 32, 16, 32
_E = 320000
_NC, _NS = 2, 16
_NW = _NC * _NS
_K = 128
_C = 160
_C0 = 112
_EPAD = _NS * _C * _K
_NP = 10112
_RPT = _NP // _NS


@functools.cache
def _make_seg_sum():
    mesh = plsc.VectorSubcoreMesh(core_axis_name="c", subcore_axis_name="s",
                                  num_cores=_NC, num_subcores=_NS)

    @functools.partial(
        pl.kernel,
        mesh=mesh,
        out_type=jax.ShapeDtypeStruct((_NC, _NP, _D), jnp.float32),
        scratch_types=[
            pltpu.VMEM((_C0, _K), jnp.int32),
            pltpu.VMEM((_C0, _K), jnp.int32),
            pltpu.VMEM((_K, _D), jnp.float32),
            pltpu.VMEM_SHARED((_NP, _D), jnp.float32),
            pltpu.SemaphoreType.DMA,
        ],
    )
    def seg_sum(x_hbm, src_hbm, dst_hbm, zero_hbm, out_hbm,
                src_v, dst_v, rows_v, acc, sem):
        c = lax.axis_index("c")
        s = lax.axis_index("s")
        r0 = s * _RPT
        pltpu.sync_copy(zero_hbm.at[pl.ds(r0, _RPT)], acc.at[pl.ds(r0, _RPT)])
        off = c * (_C - _C0)
        js = c * (2 * _C0 - _C)
        pltpu.sync_copy(src_hbm.at[s].at[pl.ds(off, _C0)], src_v)
        pltpu.sync_copy(dst_hbm.at[s].at[pl.ds(off, _C0)], dst_v)
        plsc.subcore_barrier()

        def body(j, carry):
            @pl.when(j >= js)
            def _():
                pltpu.async_copy(x_hbm.at[src_v.at[j]], rows_v, sem).wait()
                pltpu.sync_copy(rows_v, acc.at[dst_v.at[j]], add=True)
            return carry

        lax.fori_loop(0, _C0, body, 0)
        plsc.subcore_barrier()
        pltpu.sync_copy(acc.at[pl.ds(r0, _RPT)],
                        out_hbm.at[c].at[pl.ds(r0, _RPT)])

    return seg_sum


def _seg_sum(x_pad, srcp, dstp, zero_np):
    return _make_seg_sum()(x_pad, srcp, dstp, zero_np)


def _elu(v):
    return jnp.where(v > 0, v, jnp.exp(jnp.minimum(v, 0.0)) - 1.0)


def _bn(y, g, be):
    m = jnp.mean(y, axis=0, keepdims=True)
    v = jnp.mean((y - m) ** 2, axis=0, keepdims=True)
    return (y - m) / jnp.sqrt(v + 1e-5) * g + be


def _lin_block(h, W1, b1, g1, be1, W2, b2, g2, be2):
    a = jnp.dot(h, W1, preferred_element_type=jnp.float32) + b1
    e = _elu(_bn(a, g1, be1))
    cc = jnp.dot(e, W2, preferred_element_type=jnp.float32) + b2
    return _bn(cc, g2, be2)


def _dense1_body(x_ref, agg_ref, Wg, bg, gg, beg,
                 W1, b1, g1, be1, W2, b2, g2, be2,
                 h1_ref, z0_ref):
    x = x_ref[...]
    h2 = x + agg_ref[0, :_N, :] + agg_ref[1, :_N, :]
    y = jnp.dot(h2, Wg[...], preferred_element_type=jnp.float32) + bg[...]
    h1_ref[:_N, :] = _elu(_bn(y, gg[...], beg[...]))
    h1_ref[_N:, :] = jnp.zeros((_NP - _N, _D), jnp.float32)
    z0_ref[...] = _lin_block(x, W1[...], b1[...], g1[...], be1[...],
                             W2[...], b2[...], g2[...], be2[...])


_dense1 = pl.pallas_call(
    _dense1_body,
    out_shape=[
        jax.ShapeDtypeStruct((_NP, _D), jnp.float32),
        jax.ShapeDtypeStruct((_N, _T), jnp.float32),
    ],
)


def _dense2_body(h1p_ref, agg_ref, z0_ref, batch_ref,
                 Wg, bg, gg, beg,
                 aW1, ab1, ag1, abe1, aW2, ab2, ag2, abe2,
                 bW1, bb1, bg1, bbe1, bW2, bb2, bg2, bbe2,
                 out_ref, Z_ref, h_ref):
    h1 = h1p_ref[:_N, :]
    hpre = h1 + agg_ref[0, :_N, :] + agg_ref[1, :_N, :]
    y = jnp.dot(hpre, Wg[...], preferred_element_type=jnp.float32) + bg[...]
    hh = _elu(_bn(y, gg[...], beg[...]))
    h_ref[...] = hh
    z1 = _lin_block(h1, aW1[...], ab1[...], ag1[...], abe1[...],
                    aW2[...], ab2[...], ag2[...], abe2[...])
    z2 = _lin_block(hh, bW1[...], bb1[...], bg1[...], bbe1[...],
                    bW2[...], bb2[...], bg2[...], bbe2[...])
    z0 = z0_ref[...]
    Z_ref[...] = z0 + z1 + z2
    b = batch_ref[...]
    ninf = jnp.float32(-jnp.inf)
    zc = jnp.concatenate([z0, z1, z2], axis=1)

    def pool_body(g, carry):
        m = b == g
        p = jnp.max(jnp.where(m, zc, ninf), axis=0, keepdims=True)
        out_ref[pl.ds(g, 1), :] = p[:, 0:_T] + p[:, _T:2 * _T] + p[:, 2 * _T:]
        return carry

    lax.fori_loop(0, _G, pool_body, 0)


_dense2 = pl.pallas_call(
    _dense2_body,
    out_shape=[
        jax.ShapeDtypeStruct((_G, _T), jnp.float32),
        jax.ShapeDtypeStruct((_N, _T), jnp.float32),
        jax.ShapeDtypeStruct((_N, _D), jnp.float32),
    ],
)


def _flat_lin(p):
    return (p["W1"], p["b1"].reshape(1, -1), p["g1"].reshape(1, -1),
            p["be1"].reshape(1, -1), p["W2"], p["b2"].reshape(1, -1),
            p["g2"].reshape(1, -1), p["be2"].reshape(1, -1))


def _flat_gin(p):
    return (p["W"], p["b"].reshape(1, -1), p["g"].reshape(1, -1),
            p["be"].reshape(1, -1))


def kernel(x, params, edge_index, batch):
    pad = _EPAD - _E
    srcp = jnp.concatenate(
        [edge_index[0], jnp.full((pad,), _N, jnp.int32)]).reshape(_NS, _C, _K)
    dstp = jnp.concatenate(
        [edge_index[1], jnp.full((pad,), _N, jnp.int32)]).reshape(_NS, _C, _K)
    zero_np = jnp.zeros((_NP, _D), jnp.float32)
    x_pad = jnp.concatenate(
        [x, jnp.zeros((_NP - _N, _D), jnp.float32)], axis=0)

    agg1 = _seg_sum(x_pad, srcp, dstp, zero_np)
    h1p, z0 = _dense1(x, agg1, *_flat_gin(params["gin1"]),
                      *_flat_lin(params["lin0"]))
    agg2 = _seg_sum(h1p, srcp, dstp, zero_np)
    out, Z, h = _dense2(h1p, agg2, z0, batch.reshape(_N, 1),
                        *_flat_gin(params["gin2"]),
                        *_flat_lin(params["lin1"]),
                        *_flat_lin(params["lin2"]))
    return out, Z, h


# speedup vs baseline: 3.6136x; 1.0150x over previous
import functools

import jax
import jax.numpy as jnp
from jax import lax
from jax.experimental import pallas as pl
from jax.experimental.pallas import tpu as pltpu
from jax.experimental.pallas import tpu_sc as plsc

_N, _D, _FD, _T, _G = 10000, 128, 32, 16, 32
_E = 320000
_NC, _NS = 2, 16
_NW = _NC * _NS
_K = 128
_C = 160
_C0 = 120
_EPAD = _NS * _C * _K
_NP = 10112
_RPT = _NP // _NS


@functools.cache
def _make_seg_sum():
    mesh = plsc.VectorSubcoreMesh(core_axis_name="c", subcore_axis_name="s",
                                  num_cores=_NC, num_subcores=_NS)

    @functools.partial(
        pl.kernel,
        mesh=mesh,
        out_type=jax.ShapeDtypeStruct((_NC, _NP, _D), jnp.float32),
        scratch_types=[
            pltpu.VMEM((_C0, _K), jnp.int32),
            pltpu.VMEM((_C0, _K), jnp.int32),
            pltpu.VMEM((_K, _D), jnp.float32),
            pltpu.VMEM_SHARED((_NP, _D), jnp.float32),
            pltpu.SemaphoreType.DMA,
        ],
    )
    def seg_sum(x_hbm, src_hbm, dst_hbm, zero_hbm, out_hbm,
                src_v, dst_v, rows_v, acc, sem):
        c = lax.axis_index("c")
        s = lax.axis_index("s")
        r0 = s * _RPT
        pltpu.sync_copy(zero_hbm.at[pl.ds(r0, _RPT)], acc.at[pl.ds(r0, _RPT)])
        off = c * (_C - _C0)
        js = c * (2 * _C0 - _C)
        pltpu.sync_copy(src_hbm.at[s].at[pl.ds(off, _C0)], src_v)
        pltpu.sync_copy(dst_hbm.at[s].at[pl.ds(off, _C0)], dst_v)
        plsc.subcore_barrier()

        def body(j, carry):
            @pl.when(j >= js)
            def _():
                pltpu.async_copy(x_hbm.at[src_v.at[j]], rows_v, sem).wait()
                pltpu.sync_copy(rows_v, acc.at[dst_v.at[j]], add=True)
            return carry

        lax.fori_loop(0, _C0, body, 0)
        plsc.subcore_barrier()
        pltpu.sync_copy(acc.at[pl.ds(r0, _RPT)],
                        out_hbm.at[c].at[pl.ds(r0, _RPT)])

    return seg_sum


def _seg_sum(x_pad, srcp, dstp, zero_np):
    return _make_seg_sum()(x_pad, srcp, dstp, zero_np)


def _elu(v):
    return jnp.where(v > 0, v, jnp.exp(jnp.minimum(v, 0.0)) - 1.0)


def _bn(y, g, be):
    m = jnp.mean(y, axis=0, keepdims=True)
    v = jnp.mean((y - m) ** 2, axis=0, keepdims=True)
    return (y - m) / jnp.sqrt(v + 1e-5) * g + be


def _lin_block(h, W1, b1, g1, be1, W2, b2, g2, be2):
    a = jnp.dot(h, W1, preferred_element_type=jnp.float32) + b1
    e = _elu(_bn(a, g1, be1))
    cc = jnp.dot(e, W2, preferred_element_type=jnp.float32) + b2
    return _bn(cc, g2, be2)


def _dense1a_body(x_ref, agg_ref, Wg, bg, gg, beg, h1_ref):
    x = x_ref[...]
    h2 = x + agg_ref[0, :_N, :] + agg_ref[1, :_N, :]
    y = jnp.dot(h2, Wg[...], preferred_element_type=jnp.float32) + bg[...]
    h1_ref[:_N, :] = _elu(_bn(y, gg[...], beg[...]))
    h1_ref[_N:, :] = jnp.zeros((_NP - _N, _D), jnp.float32)


_dense1a = pl.pallas_call(
    _dense1a_body,
    out_shape=jax.ShapeDtypeStruct((_NP, _D), jnp.float32),
)


def _dense1b_body(x_ref, h1p_ref, batch_ref,
                 aW1, ab1, ag1, abe1, aW2, ab2, ag2, abe2,
                 bW1, bb1, bg1, bbe1, bW2, bb2, bg2, bbe2,
                 z0_ref, z1_ref, p01_ref):
    x = x_ref[...]
    z0 = _lin_block(x, aW1[...], ab1[...], ag1[...], abe1[...],
                    aW2[...], ab2[...], ag2[...], abe2[...])
    z0_ref[...] = z0
    z1 = _lin_block(h1p_ref[:_N, :], bW1[...], bb1[...], bg1[...], bbe1[...],
                    bW2[...], bb2[...], bg2[...], bbe2[...])
    z1_ref[...] = z1
    b = batch_ref[...]
    ninf = jnp.float32(-jnp.inf)
    zc = jnp.concatenate([z0, z1], axis=1)

    def pool_body(g, carry):
        m = b == g
        p = jnp.max(jnp.where(m, zc, ninf), axis=0, keepdims=True)
        p01_ref[pl.ds(g, 1), :] = p[:, 0:_T] + p[:, _T:]
        return carry

    lax.fori_loop(0, _G, pool_body, 0)


_dense1b = pl.pallas_call(
    _dense1b_body,
    out_shape=[
        jax.ShapeDtypeStruct((_N, _T), jnp.float32),
        jax.ShapeDtypeStruct((_N, _T), jnp.float32),
        jax.ShapeDtypeStruct((_G, _T), jnp.float32),
    ],
)


def _dense2_body(h1p_ref, agg_ref, z0_ref, z1_ref, p01_ref, batch_ref,
                 Wg, bg, gg, beg,
                 bW1, bb1, bg1, bbe1, bW2, bb2, bg2, bbe2,
                 out_ref, Z_ref, h_ref):
    h1 = h1p_ref[:_N, :]
    hpre = h1 + agg_ref[0, :_N, :] + agg_ref[1, :_N, :]
    y = jnp.dot(hpre, Wg[...], preferred_element_type=jnp.float32) + bg[...]
    hh = _elu(_bn(y, gg[...], beg[...]))
    h_ref[...] = hh
    z2 = _lin_block(hh, bW1[...], bb1[...], bg1[...], bbe1[...],
                    bW2[...], bb2[...], bg2[...], bbe2[...])
    Z_ref[...] = z0_ref[...] + z1_ref[...] + z2
    b = batch_ref[...]
    ninf = jnp.float32(-jnp.inf)

    def pool_body(g, carry):
        m = b == g
        p = jnp.max(jnp.where(m, z2, ninf), axis=0, keepdims=True)
        out_ref[pl.ds(g, 1), :] = p01_ref[pl.ds(g, 1), :] + p
        return carry

    lax.fori_loop(0, _G, pool_body, 0)


_dense2 = pl.pallas_call(
    _dense2_body,
    out_shape=[
        jax.ShapeDtypeStruct((_G, _T), jnp.float32),
        jax.ShapeDtypeStruct((_N, _T), jnp.float32),
        jax.ShapeDtypeStruct((_N, _D), jnp.float32),
    ],
)


def _flat_lin(p):
    return (p["W1"], p["b1"].reshape(1, -1), p["g1"].reshape(1, -1),
            p["be1"].reshape(1, -1), p["W2"], p["b2"].reshape(1, -1),
            p["g2"].reshape(1, -1), p["be2"].reshape(1, -1))


def _flat_gin(p):
    return (p["W"], p["b"].reshape(1, -1), p["g"].reshape(1, -1),
            p["be"].reshape(1, -1))


def kernel(x, params, edge_index, batch):
    pad = _EPAD - _E
    srcp = jnp.concatenate(
        [edge_index[0], jnp.full((pad,), _N, jnp.int32)]).reshape(_NS, _C, _K)
    dstp = jnp.concatenate(
        [edge_index[1], jnp.full((pad,), _N, jnp.int32)]).reshape(_NS, _C, _K)
    zero_np = jnp.zeros((_NP, _D), jnp.float32)
    x_pad = jnp.concatenate(
        [x, jnp.zeros((_NP - _N, _D), jnp.float32)], axis=0)

    agg1 = _seg_sum(x_pad, srcp, dstp, zero_np)
    h1p = _dense1a(x, agg1, *_flat_gin(params["gin1"]))
    agg2 = _seg_sum(h1p, srcp, dstp, zero_np)
    b2d = batch.reshape(_N, 1)
    z0, z1, p01 = _dense1b(x, h1p, b2d,
                           *_flat_lin(params["lin0"]),
                           *_flat_lin(params["lin1"]))
    out, Z, h = _dense2(h1p, agg2, z0, z1, p01, b2d,
                        *_flat_gin(params["gin2"]),
                        *_flat_lin(params["lin2"]))
    return out, Z, h
